# Initial kernel scaffold; baseline (speedup 1.0000x reference)
import jax
import jax.numpy as jnp
from jax import lax
from jax.experimental import pallas as pl
from jax.experimental.pallas import tpu as pltpu
from jax.experimental.pallas import tpu_sc as plsc

N = 10000
D = 128
E = 320000
H = 16

NC = 2
NS = 16
CHUNK = 80
EDGES_PER_TILE = E // (NC * NS)
NCHUNK = EDGES_PER_TILE // CHUNK
SLAB = N // NS


def _row_conv_body(edge_hbm, w_hbm, table_hbm, out_hbm,
                   acc_sh, row_v, col_v, w_v, msg_v, slab_v, sem):
    c = lax.axis_index("c")
    s = lax.axis_index("s")
    tile_base = (c * NS + s) * EDGES_PER_TILE

    def zbody(i, _):
        slab_v[i, :] = jnp.zeros((H,), jnp.float32)
        return 0
    lax.fori_loop(0, SLAB, zbody, 0)
    pltpu.sync_copy(slab_v, acc_sh.at[pl.ds(s * SLAB, SLAB)])
    plsc.subcore_barrier()

    iota16 = lax.iota(jnp.int32, 16)

    def body(i, _):
        base = pl.multiple_of(tile_base + i * CHUNK, 8)
        pltpu.sync_copy(edge_hbm.at[0, pl.ds(base, CHUNK)], row_v)
        pltpu.sync_copy(edge_hbm.at[1, pl.ds(base, CHUNK)], col_v)
        pltpu.sync_copy(w_hbm.at[pl.ds(base, CHUNK)], w_v)
        pltpu.async_copy(table_hbm.at[row_v], msg_v, sem).wait()
        for g in range(CHUNK // 16):
            wv = w_v[pl.ds(g * 16, 16)]
            ev = iota16 + g * 16
            for h in range(H):
                hv = jnp.full((16,), h, jnp.int32)
                v = plsc.load_gather(msg_v, [ev, hv])
                plsc.store_scatter(msg_v, [ev, hv], v * wv)
        pltpu.sync_copy(msg_v, acc_sh.at[col_v], add=True)
        return 0

    lax.fori_loop(0, NCHUNK, body, 0)
    plsc.subcore_barrier()

    pltpu.sync_copy(acc_sh.at[pl.ds(s * SLAB, SLAB)], slab_v)
    pltpu.sync_copy(slab_v, out_hbm.at[c, pl.ds(s * SLAB, SLAB)])


def _row_conv(edge_index, weight, table):
    mesh = plsc.VectorSubcoreMesh(core_axis_name="c", subcore_axis_name="s")
    f = pl.kernel(
        _row_conv_body,
        out_type=jax.ShapeDtypeStruct((NC, N, H), jnp.float32),
        mesh=mesh,
        scratch_types=[
            pltpu.VMEM_SHARED((N, H), jnp.float32),
            pltpu.VMEM((CHUNK,), jnp.int32),
            pltpu.VMEM((CHUNK,), jnp.int32),
            pltpu.VMEM((CHUNK,), jnp.float32),
            pltpu.VMEM((CHUNK, H), jnp.float32),
            pltpu.VMEM((SLAB, H), jnp.float32),
            pltpu.SemaphoreType.DMA,
        ],
    )
    return f(edge_index, weight, table)


def _scalar_conv_body(edge_hbm, w_hbm, table_hbm, out_hbm,
                      acc_sh, tbl_v, row_v, col_v, w_v, res_v, full_v, sem):
    c = lax.axis_index("c")
    s = lax.axis_index("s")
    tile_base = (c * NS + s) * EDGES_PER_TILE

    pltpu.sync_copy(table_hbm, tbl_v)

    @pl.when(s == 0)
    def _():
        def zbody(i, _):
            full_v[pl.ds(i * 16, 16)] = jnp.zeros((16,), jnp.float32)
            return 0
        lax.fori_loop(0, N // 16, zbody, 0)
        pltpu.sync_copy(full_v, acc_sh)
    plsc.subcore_barrier()

    def body(i, _):
        base = pl.multiple_of(tile_base + i * CHUNK, 8)
        pltpu.sync_copy(edge_hbm.at[0, pl.ds(base, CHUNK)], row_v)
        pltpu.sync_copy(edge_hbm.at[1, pl.ds(base, CHUNK)], col_v)
        pltpu.sync_copy(w_hbm.at[pl.ds(base, CHUNK)], w_v)
        for g in range(CHUNK // 16):
            rv = row_v[pl.ds(g * 16, 16)]
            wv = w_v[pl.ds(g * 16, 16)]
            res_v[pl.ds(g * 16, 16)] = plsc.load_gather(tbl_v, [rv]) * wv
        pltpu.sync_copy(res_v, acc_sh.at[col_v], add=True)
        return 0

    lax.fori_loop(0, NCHUNK, body, 0)
    plsc.subcore_barrier()

    @pl.when(s == 0)
    def _():
        pltpu.sync_copy(acc_sh, full_v)
        pltpu.sync_copy(full_v, out_hbm.at[c])


def _scalar_conv(edge_index, weight, table):
    mesh = plsc.VectorSubcoreMesh(core_axis_name="c", subcore_axis_name="s")
    f = pl.kernel(
        _scalar_conv_body,
        out_type=jax.ShapeDtypeStruct((NC, N), jnp.float32),
        mesh=mesh,
        scratch_types=[
            pltpu.VMEM_SHARED((N,), jnp.float32),
            pltpu.VMEM((N,), jnp.float32),
            pltpu.VMEM((CHUNK,), jnp.int32),
            pltpu.VMEM((CHUNK,), jnp.int32),
            pltpu.VMEM((CHUNK,), jnp.float32),
            pltpu.VMEM((CHUNK,), jnp.float32),
            pltpu.VMEM((N,), jnp.float32),
            pltpu.SemaphoreType.DMA,
        ],
    )
    return f(edge_index, weight, table)


def kernel(x, edge_index, weight, W1, b1, W2, b2, W3, b3, A2w, A2b):
    ones_n = jnp.ones((N,), jnp.float32)

    degp = _scalar_conv(edge_index, weight, ones_n)
    deg = degp[0] + degp[1] + 1.0
    dinv = jnp.where(deg > 0, deg ** -0.5, 0.0)
    dinv2 = dinv * dinv

    def conv_epilogue(partials, hw, b):
        return (partials[0] + partials[1]) * dinv[:, None] \
            + hw * dinv2[:, None] + b

    hw1 = x @ W1
    p1 = _row_conv(edge_index, weight, hw1 * dinv[:, None])
    h1 = jax.nn.relu(conv_epilogue(p1, hw1, b1))

    hw2 = h1 @ W2
    p2 = _row_conv(edge_index, weight, hw2 * dinv[:, None])
    h2 = jax.nn.relu(conv_epilogue(p2, hw2, b2))

    hw3 = (h2 @ W3)[:, 0]
    p3 = _scalar_conv(edge_index, weight, hw3 * dinv)
    c = (p3[0] + p3[1]) * dinv + hw3 * dinv2 + b3[0]

    choice = jax.nn.softmax(c, axis=0)
    v = jnp.mean(h2, axis=0, keepdims=True)
    value = (v @ A2w.T + A2b).squeeze()
    return (choice, value)


# speedup vs baseline: 10.7356x; 10.7356x over previous
import jax
import jax.numpy as jnp
from jax import lax
from jax.experimental import pallas as pl
from jax.experimental.pallas import tpu as pltpu
from jax.experimental.pallas import tpu_sc as plsc

N = 10000
D = 128
E = 320000
H = 16

NC = 2
NS = 16
CHUNK = 80
EDGES_PER_TILE = E // (NC * NS)
NCHUNK = EDGES_PER_TILE // CHUNK
SLAB = 640
SLAB_LAST = N - 15 * SLAB


def _row_conv_body(row_hbm, col_hbm, w_hbm, table_hbm, out_hbm,
                   acc_sh, row_v, col_v, w_v, msg_v, slab_v, sem):
    c = lax.axis_index("c")
    s = lax.axis_index("s")
    tile_base = (c * NS + s) * EDGES_PER_TILE

    def zbody(i, _):
        slab_v[i, :] = jnp.zeros((H,), jnp.float32)
        return 0
    lax.fori_loop(0, SLAB, zbody, 0)

    @pl.when(s < 15)
    def _():
        pltpu.sync_copy(slab_v, acc_sh.at[pl.ds(s * SLAB, SLAB)])

    @pl.when(s == 15)
    def _():
        pltpu.sync_copy(slab_v.at[pl.ds(0, SLAB_LAST)],
                        acc_sh.at[pl.ds(15 * SLAB, SLAB_LAST)])
    plsc.subcore_barrier()

    iota16 = lax.iota(jnp.int32, 16)

    def body(i, _):
        base = pl.multiple_of(tile_base + i * CHUNK, 8)
        pltpu.sync_copy(row_hbm.at[pl.ds(base, CHUNK)], row_v)
        pltpu.sync_copy(col_hbm.at[pl.ds(base, CHUNK)], col_v)
        pltpu.sync_copy(w_hbm.at[pl.ds(base, CHUNK)], w_v)
        pltpu.async_copy(table_hbm.at[row_v], msg_v, sem).wait()
        for g in range(CHUNK // 16):
            wv = w_v[pl.ds(g * 16, 16)]
            ev = iota16 + g * 16
            for h in range(H):
                hv = jnp.full((16,), h, jnp.int32)
                v = plsc.load_gather(msg_v, [ev, hv])
                plsc.store_scatter(msg_v, [ev, hv], v * wv)
        pltpu.sync_copy(msg_v, acc_sh.at[col_v], add=True)
        return 0

    lax.fori_loop(0, NCHUNK, body, 0)
    plsc.subcore_barrier()

    @pl.when(s < 15)
    def _():
        pltpu.sync_copy(acc_sh.at[pl.ds(s * SLAB, SLAB)], slab_v)
        pltpu.sync_copy(slab_v, out_hbm.at[c, pl.ds(s * SLAB, SLAB)])

    @pl.when(s == 15)
    def _():
        pltpu.sync_copy(acc_sh.at[pl.ds(15 * SLAB, SLAB_LAST)],
                        slab_v.at[pl.ds(0, SLAB_LAST)])
        pltpu.sync_copy(slab_v.at[pl.ds(0, SLAB_LAST)],
                        out_hbm.at[c, pl.ds(15 * SLAB, SLAB_LAST)])


def _row_conv(row, col, weight, table):
    mesh = plsc.VectorSubcoreMesh(core_axis_name="c", subcore_axis_name="s")
    f = pl.kernel(
        _row_conv_body,
        out_type=jax.ShapeDtypeStruct((NC, N, H), jnp.float32),
        mesh=mesh,
        compiler_params=pltpu.CompilerParams(needs_layout_passes=False, use_tc_tiling_on_sc=False),
        scratch_types=[
            pltpu.VMEM_SHARED((N, H), jnp.float32),
            pltpu.VMEM((CHUNK,), jnp.int32),
            pltpu.VMEM((CHUNK,), jnp.int32),
            pltpu.VMEM((CHUNK,), jnp.float32),
            pltpu.VMEM((CHUNK, H), jnp.float32),
            pltpu.VMEM((SLAB, H), jnp.float32),
            pltpu.SemaphoreType.DMA,
        ],
    )
    return f(row, col, weight, table)


def _scalar_conv_body(row_hbm, col_hbm, w_hbm, table_hbm, out_hbm,
                      acc_sh, tbl_v, row_v, col_v, w_v, res_v, full_v, sem):
    c = lax.axis_index("c")
    s = lax.axis_index("s")
    tile_base = (c * NS + s) * EDGES_PER_TILE

    pltpu.sync_copy(table_hbm, tbl_v)

    @pl.when(s == 0)
    def _():
        def zbody(i, _):
            full_v[pl.ds(i * 16, 16)] = jnp.zeros((16,), jnp.float32)
            return 0
        lax.fori_loop(0, N // 16, zbody, 0)
        pltpu.sync_copy(full_v, acc_sh)
    plsc.subcore_barrier()

    def body(i, _):
        base = pl.multiple_of(tile_base + i * CHUNK, 8)
        pltpu.sync_copy(row_hbm.at[pl.ds(base, CHUNK)], row_v)
        pltpu.sync_copy(col_hbm.at[pl.ds(base, CHUNK)], col_v)
        pltpu.sync_copy(w_hbm.at[pl.ds(base, CHUNK)], w_v)
        for g in range(CHUNK // 16):
            rv = row_v[pl.ds(g * 16, 16)]
            wv = w_v[pl.ds(g * 16, 16)]
            res_v[pl.ds(g * 16, 16)] = plsc.load_gather(tbl_v, [rv]) * wv
        pltpu.sync_copy(res_v, acc_sh.at[col_v], add=True)
        return 0

    lax.fori_loop(0, NCHUNK, body, 0)
    plsc.subcore_barrier()

    @pl.when(s == 0)
    def _():
        pltpu.sync_copy(acc_sh, full_v)
        pltpu.sync_copy(full_v, out_hbm.at[c])


def _scalar_conv(row, col, weight, table):
    mesh = plsc.VectorSubcoreMesh(core_axis_name="c", subcore_axis_name="s")
    f = pl.kernel(
        _scalar_conv_body,
        out_type=jax.ShapeDtypeStruct((NC, N), jnp.float32),
        mesh=mesh,
        compiler_params=pltpu.CompilerParams(needs_layout_passes=False, use_tc_tiling_on_sc=False),
        scratch_types=[
            pltpu.VMEM_SHARED((N,), jnp.float32),
            pltpu.VMEM((N,), jnp.float32),
            pltpu.VMEM((CHUNK,), jnp.int32),
            pltpu.VMEM((CHUNK,), jnp.int32),
            pltpu.VMEM((CHUNK,), jnp.float32),
            pltpu.VMEM((CHUNK,), jnp.float32),
            pltpu.VMEM((N,), jnp.float32),
            pltpu.SemaphoreType.DMA,
        ],
    )
    return f(row, col, weight, table)


def kernel(x, edge_index, weight, W1, b1, W2, b2, W3, b3, A2w, A2b):
    ones_n = jnp.ones((N,), jnp.float32)
    row = edge_index[0]
    col = edge_index[1]

    degp = _scalar_conv(row, col, weight, ones_n)
    deg = degp[0] + degp[1] + 1.0
    dinv = jnp.where(deg > 0, deg ** -0.5, 0.0)
    dinv2 = dinv * dinv

    def conv_epilogue(partials, hw, b):
        return (partials[0] + partials[1]) * dinv[:, None] \
            + hw * dinv2[:, None] + b

    hw1 = x @ W1
    p1 = _row_conv(row, col, weight, hw1 * dinv[:, None])
    h1 = jax.nn.relu(conv_epilogue(p1, hw1, b1))

    hw2 = h1 @ W2
    p2 = _row_conv(row, col, weight, hw2 * dinv[:, None])
    h2 = jax.nn.relu(conv_epilogue(p2, hw2, b2))

    hw3 = (h2 @ W3)[:, 0]
    p3 = _scalar_conv(row, col, weight, hw3 * dinv)
    c = (p3[0] + p3[1]) * dinv + hw3 * dinv2 + b3[0]

    choice = jax.nn.softmax(c, axis=0)
    v = jnp.mean(h2, axis=0, keepdims=True)
    value = (v @ A2w.T + A2b).squeeze()
    return (choice, value)


# speedup vs baseline: 22.7886x; 2.1227x over previous
import jax
import jax.numpy as jnp
from jax import lax
from jax.experimental import pallas as pl
from jax.experimental.pallas import tpu as pltpu
from jax.experimental.pallas import tpu_sc as plsc

N = 10000
D = 128
E = 320000
H = 16

NC = 2
NS = 16
CHUNK = 400
EDGES_PER_TILE = E // (NC * NS)
NCHUNK = EDGES_PER_TILE // CHUNK
SLAB = 640
SLAB_LAST = N - 15 * SLAB


def _row_conv_body(row_hbm, col_hbm, w_hbm, table_hbm, out_hbm,
                   acc_sh, row_v, col_v, w_v, msg_v, slab_v, sem):
    c = lax.axis_index("c")
    s = lax.axis_index("s")
    tile_base = (c * NS + s) * EDGES_PER_TILE

    def zbody(i, _):
        slab_v[i, :] = jnp.zeros((H,), jnp.float32)
        return 0
    lax.fori_loop(0, SLAB, zbody, 0)

    @pl.when(s < 15)
    def _():
        pltpu.sync_copy(slab_v, acc_sh.at[pl.ds(s * SLAB, SLAB)])

    @pl.when(s == 15)
    def _():
        pltpu.sync_copy(slab_v.at[pl.ds(0, SLAB_LAST)],
                        acc_sh.at[pl.ds(15 * SLAB, SLAB_LAST)])
    plsc.subcore_barrier()

    iota16 = lax.iota(jnp.int32, 16)

    def body(i, _):
        base = pl.multiple_of(tile_base + i * CHUNK, 8)
        pltpu.sync_copy(row_hbm.at[pl.ds(base, CHUNK)], row_v)
        pltpu.sync_copy(col_hbm.at[pl.ds(base, CHUNK)], col_v)
        pltpu.sync_copy(w_hbm.at[pl.ds(base, CHUNK)], w_v)
        pltpu.async_copy(table_hbm.at[row_v], msg_v, sem).wait()
        for g in range(CHUNK // 16):
            wv = w_v[pl.ds(g * 16, 16)]
            ev = iota16 + g * 16
            for h in range(H):
                hv = jnp.full((16,), h, jnp.int32)
                v = plsc.load_gather(msg_v, [ev, hv])
                plsc.store_scatter(msg_v, [ev, hv], v * wv)
        pltpu.sync_copy(msg_v, acc_sh.at[col_v], add=True)
        return 0

    lax.fori_loop(0, NCHUNK, body, 0)
    plsc.subcore_barrier()

    @pl.when(s < 15)
    def _():
        pltpu.sync_copy(acc_sh.at[pl.ds(s * SLAB, SLAB)], slab_v)
        pltpu.sync_copy(slab_v, out_hbm.at[c, pl.ds(s * SLAB, SLAB)])

    @pl.when(s == 15)
    def _():
        pltpu.sync_copy(acc_sh.at[pl.ds(15 * SLAB, SLAB_LAST)],
                        slab_v.at[pl.ds(0, SLAB_LAST)])
        pltpu.sync_copy(slab_v.at[pl.ds(0, SLAB_LAST)],
                        out_hbm.at[c, pl.ds(15 * SLAB, SLAB_LAST)])


def _row_conv(row, col, weight, table):
    mesh = plsc.VectorSubcoreMesh(core_axis_name="c", subcore_axis_name="s")
    f = pl.kernel(
        _row_conv_body,
        out_type=jax.ShapeDtypeStruct((NC, N, H), jnp.float32),
        mesh=mesh,
        compiler_params=pltpu.CompilerParams(needs_layout_passes=False, use_tc_tiling_on_sc=False),
        scratch_types=[
            pltpu.VMEM_SHARED((N, H), jnp.float32),
            pltpu.VMEM((CHUNK,), jnp.int32),
            pltpu.VMEM((CHUNK,), jnp.int32),
            pltpu.VMEM((CHUNK,), jnp.float32),
            pltpu.VMEM((CHUNK, H), jnp.float32),
            pltpu.VMEM((SLAB, H), jnp.float32),
            pltpu.SemaphoreType.DMA,
        ],
    )
    return f(row, col, weight, table)


def _scalar_conv_body(row_hbm, col_hbm, w_hbm, table_hbm, out_hbm,
                      acc_sh, tbl_v, row_v, col_v, w_v, res_v, full_v, sem):
    c = lax.axis_index("c")
    s = lax.axis_index("s")
    tile_base = (c * NS + s) * EDGES_PER_TILE

    pltpu.sync_copy(table_hbm, tbl_v)

    @pl.when(s == 0)
    def _():
        def zbody(i, _):
            full_v[pl.ds(i * 16, 16)] = jnp.zeros((16,), jnp.float32)
            return 0
        lax.fori_loop(0, N // 16, zbody, 0)
        pltpu.sync_copy(full_v, acc_sh)
    plsc.subcore_barrier()

    def body(i, _):
        base = pl.multiple_of(tile_base + i * CHUNK, 8)
        pltpu.sync_copy(row_hbm.at[pl.ds(base, CHUNK)], row_v)
        pltpu.sync_copy(col_hbm.at[pl.ds(base, CHUNK)], col_v)
        pltpu.sync_copy(w_hbm.at[pl.ds(base, CHUNK)], w_v)
        for g in range(CHUNK // 16):
            rv = row_v[pl.ds(g * 16, 16)]
            wv = w_v[pl.ds(g * 16, 16)]
            res_v[pl.ds(g * 16, 16)] = plsc.load_gather(tbl_v, [rv]) * wv
        pltpu.sync_copy(res_v, acc_sh.at[col_v], add=True)
        return 0

    lax.fori_loop(0, NCHUNK, body, 0)
    plsc.subcore_barrier()

    @pl.when(s == 0)
    def _():
        pltpu.sync_copy(acc_sh, full_v)
        pltpu.sync_copy(full_v, out_hbm.at[c])


def _scalar_conv(row, col, weight, table):
    mesh = plsc.VectorSubcoreMesh(core_axis_name="c", subcore_axis_name="s")
    f = pl.kernel(
        _scalar_conv_body,
        out_type=jax.ShapeDtypeStruct((NC, N), jnp.float32),
        mesh=mesh,
        compiler_params=pltpu.CompilerParams(needs_layout_passes=False, use_tc_tiling_on_sc=False),
        scratch_types=[
            pltpu.VMEM_SHARED((N,), jnp.float32),
            pltpu.VMEM((N,), jnp.float32),
            pltpu.VMEM((CHUNK,), jnp.int32),
            pltpu.VMEM((CHUNK,), jnp.int32),
            pltpu.VMEM((CHUNK,), jnp.float32),
            pltpu.VMEM((CHUNK,), jnp.float32),
            pltpu.VMEM((N,), jnp.float32),
            pltpu.SemaphoreType.DMA,
        ],
    )
    return f(row, col, weight, table)


def kernel(x, edge_index, weight, W1, b1, W2, b2, W3, b3, A2w, A2b):
    ones_n = jnp.ones((N,), jnp.float32)
    row = edge_index[0]
    col = edge_index[1]

    degp = _scalar_conv(row, col, weight, ones_n)
    deg = degp[0] + degp[1] + 1.0
    dinv = jnp.where(deg > 0, deg ** -0.5, 0.0)
    dinv2 = dinv * dinv

    def conv_epilogue(partials, hw, b):
        return (partials[0] + partials[1]) * dinv[:, None] \
            + hw * dinv2[:, None] + b

    hw1 = x @ W1
    p1 = _row_conv(row, col, weight, hw1 * dinv[:, None])
    h1 = jax.nn.relu(conv_epilogue(p1, hw1, b1))

    hw2 = h1 @ W2
    p2 = _row_conv(row, col, weight, hw2 * dinv[:, None])
    h2 = jax.nn.relu(conv_epilogue(p2, hw2, b2))

    hw3 = (h2 @ W3)[:, 0]
    p3 = _scalar_conv(row, col, weight, hw3 * dinv)
    c = (p3[0] + p3[1]) * dinv + hw3 * dinv2 + b3[0]

    choice = jax.nn.softmax(c, axis=0)
    v = jnp.mean(h2, axis=0, keepdims=True)
    value = (v @ A2w.T + A2b).squeeze()
    return (choice, value)


# speedup vs baseline: 32.8527x; 1.4416x over previous
import jax
import jax.numpy as jnp
from jax import lax
from jax.experimental import pallas as pl
from jax.experimental.pallas import tpu as pltpu
from jax.experimental.pallas import tpu_sc as plsc

N = 10000
D = 128
E = 320000
H = 16

NC = 2
NS = 16
NW = NC * NS
CHUNK = 512
EPT = 10240
NCH = EPT // CHUNK
TOT = NW * NCH
EPAD = NW * EPT
SLAB = 640
SLAB_LAST = N - 15 * SLAB

_SC_PARAMS = pltpu.CompilerParams(
    needs_layout_passes=False, use_tc_tiling_on_sc=False)


def _row_conv_body(edata_hbm, table_hbm, out_hbm,
                   acc_sh, ebuf0, ebuf1, msg0, msg1, slab_v,
                   sem_e0, sem_e1, sem_g0, sem_g1):
    c = lax.axis_index("c")
    s = lax.axis_index("s")
    cbase = (c * NS + s) * NCH

    def zbody(i, _):
        slab_v[i, :] = jnp.zeros((H,), jnp.float32)
        return 0
    lax.fori_loop(0, SLAB, zbody, 0)

    @pl.when(s < 15)
    def _():
        pltpu.sync_copy(slab_v, acc_sh.at[pl.ds(s * SLAB, SLAB)])

    @pl.when(s == 15)
    def _():
        pltpu.sync_copy(slab_v.at[pl.ds(0, SLAB_LAST)],
                        acc_sh.at[pl.ds(15 * SLAB, SLAB_LAST)])
    plsc.subcore_barrier()

    iota16 = lax.iota(jnp.int32, 16)

    def start_e(ci, ebuf, sem):
        pltpu.async_copy(edata_hbm.at[ci], ebuf, sem)

    def wait_e(ci, ebuf, sem):
        pltpu.make_async_copy(edata_hbm.at[ci], ebuf, sem).wait()

    def start_g(ebuf, msg, sem):
        pltpu.async_copy(table_hbm.at[ebuf.at[0]], msg, sem)

    def wait_g(ebuf, msg, sem):
        pltpu.make_async_copy(table_hbm.at[ebuf.at[0]], msg, sem).wait()

    def scale_scatter(ebuf, msg):
        def sbody(g, _):
            wv = plsc.bitcast(ebuf[2, pl.ds(g * 16, 16)], jnp.float32)
            ev = iota16 + g * 16
            for h in range(H):
                hv = jnp.full((16,), h, jnp.int32)
                v = plsc.load_gather(msg, [ev, hv])
                plsc.store_scatter(msg, [ev, hv], v * wv)
            return 0
        lax.fori_loop(0, CHUNK // 16, sbody, 0)
        pltpu.sync_copy(msg, acc_sh.at[ebuf.at[1]], add=True)

    start_e(cbase, ebuf0, sem_e0)
    wait_e(cbase, ebuf0, sem_e0)
    start_g(ebuf0, msg0, sem_g0)
    start_e(cbase + 1, ebuf1, sem_e1)

    def body(j, _):
        i = cbase + 2 * j
        wait_e(i + 1, ebuf1, sem_e1)
        start_g(ebuf1, msg1, sem_g1)
        wait_g(ebuf0, msg0, sem_g0)
        scale_scatter(ebuf0, msg0)

        @pl.when(2 * j + 2 < NCH)
        def _():
            start_e(i + 2, ebuf0, sem_e0)

        @pl.when(2 * j + 2 < NCH)
        def _():
            wait_e(i + 2, ebuf0, sem_e0)
            start_g(ebuf0, msg0, sem_g0)
        wait_g(ebuf1, msg1, sem_g1)
        scale_scatter(ebuf1, msg1)

        @pl.when(2 * j + 3 < NCH)
        def _():
            start_e(i + 3, ebuf1, sem_e1)
        return 0

    lax.fori_loop(0, NCH // 2, body, 0)
    plsc.subcore_barrier()

    @pl.when(s < 15)
    def _():
        pltpu.sync_copy(acc_sh.at[pl.ds(s * SLAB, SLAB)], slab_v)
        pltpu.sync_copy(slab_v, out_hbm.at[c, pl.ds(s * SLAB, SLAB)])

    @pl.when(s == 15)
    def _():
        pltpu.sync_copy(acc_sh.at[pl.ds(15 * SLAB, SLAB_LAST)],
                        slab_v.at[pl.ds(0, SLAB_LAST)])
        pltpu.sync_copy(slab_v.at[pl.ds(0, SLAB_LAST)],
                        out_hbm.at[c, pl.ds(15 * SLAB, SLAB_LAST)])


def _row_conv(edata, table):
    mesh = plsc.VectorSubcoreMesh(core_axis_name="c", subcore_axis_name="s")
    f = pl.kernel(
        _row_conv_body,
        out_type=jax.ShapeDtypeStruct((NC, N, H), jnp.float32),
        mesh=mesh,
        compiler_params=_SC_PARAMS,
        scratch_types=[
            pltpu.VMEM_SHARED((N, H), jnp.float32),
            pltpu.VMEM((3, CHUNK), jnp.int32),
            pltpu.VMEM((3, CHUNK), jnp.int32),
            pltpu.VMEM((CHUNK, H), jnp.float32),
            pltpu.VMEM((CHUNK, H), jnp.float32),
            pltpu.VMEM((SLAB, H), jnp.float32),
            pltpu.SemaphoreType.DMA,
            pltpu.SemaphoreType.DMA,
            pltpu.SemaphoreType.DMA,
            pltpu.SemaphoreType.DMA,
        ],
    )
    return f(edata, table)


def _scalar_conv_body(edata_hbm, table_hbm, out_hbm,
                      acc_sh, tbl_v, ebuf0, ebuf1, res0, res1, full_v,
                      sem_e0, sem_e1):
    c = lax.axis_index("c")
    s = lax.axis_index("s")
    cbase = (c * NS + s) * NCH

    pltpu.sync_copy(table_hbm, tbl_v)

    @pl.when(s == 0)
    def _():
        def zbody(i, _):
            full_v[pl.ds(i * 16, 16)] = jnp.zeros((16,), jnp.float32)
            return 0
        lax.fori_loop(0, N // 16, zbody, 0)
        pltpu.sync_copy(full_v, acc_sh)
    plsc.subcore_barrier()

    def start_e(ci, ebuf, sem):
        pltpu.async_copy(edata_hbm.at[ci], ebuf, sem)

    def wait_e(ci, ebuf, sem):
        pltpu.make_async_copy(edata_hbm.at[ci], ebuf, sem).wait()

    def compute_scatter(ebuf, res):
        def sbody(g, _):
            rv = ebuf[0, pl.ds(g * 16, 16)]
            wv = plsc.bitcast(ebuf[2, pl.ds(g * 16, 16)], jnp.float32)
            res[pl.ds(g * 16, 16)] = plsc.load_gather(tbl_v, [rv]) * wv
            return 0
        lax.fori_loop(0, CHUNK // 16, sbody, 0)
        pltpu.sync_copy(res, acc_sh.at[ebuf.at[1]], add=True)

    start_e(cbase, ebuf0, sem_e0)

    def body(j, _):
        i = cbase + 2 * j
        wait_e(i, ebuf0, sem_e0)
        start_e(i + 1, ebuf1, sem_e1)
        compute_scatter(ebuf0, res0)
        wait_e(i + 1, ebuf1, sem_e1)

        @pl.when(2 * j + 2 < NCH)
        def _():
            start_e(i + 2, ebuf0, sem_e0)
        compute_scatter(ebuf1, res1)
        return 0

    lax.fori_loop(0, NCH // 2, body, 0)
    plsc.subcore_barrier()

    @pl.when(s == 0)
    def _():
        pltpu.sync_copy(acc_sh, full_v)
        pltpu.sync_copy(full_v, out_hbm.at[c])


def _scalar_conv(edata, table):
    mesh = plsc.VectorSubcoreMesh(core_axis_name="c", subcore_axis_name="s")
    f = pl.kernel(
        _scalar_conv_body,
        out_type=jax.ShapeDtypeStruct((NC, N), jnp.float32),
        mesh=mesh,
        compiler_params=_SC_PARAMS,
        scratch_types=[
            pltpu.VMEM_SHARED((N,), jnp.float32),
            pltpu.VMEM((N,), jnp.float32),
            pltpu.VMEM((3, CHUNK), jnp.int32),
            pltpu.VMEM((3, CHUNK), jnp.int32),
            pltpu.VMEM((CHUNK,), jnp.float32),
            pltpu.VMEM((CHUNK,), jnp.float32),
            pltpu.VMEM((N,), jnp.float32),
            pltpu.SemaphoreType.DMA,
            pltpu.SemaphoreType.DMA,
        ],
    )
    return f(edata, table)


def kernel(x, edge_index, weight, W1, b1, W2, b2, W3, b3, A2w, A2b):
    pad = EPAD - E
    rowp = jnp.concatenate([edge_index[0], jnp.zeros((pad,), jnp.int32)])
    colp = jnp.concatenate([edge_index[1], jnp.zeros((pad,), jnp.int32)])
    wp = jnp.concatenate([weight, jnp.zeros((pad,), jnp.float32)])
    edata = jnp.stack(
        [rowp, colp, lax.bitcast_convert_type(wp, jnp.int32)], axis=0)
    edata = edata.reshape(3, TOT, CHUNK).transpose(1, 0, 2)

    ones_n = jnp.ones((N,), jnp.float32)

    degp = _scalar_conv(edata, ones_n)
    deg = degp[0] + degp[1] + 1.0
    dinv = jnp.where(deg > 0, deg ** -0.5, 0.0)
    dinv2 = dinv * dinv

    def conv_epilogue(partials, hw, b):
        return (partials[0] + partials[1]) * dinv[:, None] \
            + hw * dinv2[:, None] + b

    hw1 = x @ W1
    p1 = _row_conv(edata, hw1 * dinv[:, None])
    h1 = jax.nn.relu(conv_epilogue(p1, hw1, b1))

    hw2 = h1 @ W2
    p2 = _row_conv(edata, hw2 * dinv[:, None])
    h2 = jax.nn.relu(conv_epilogue(p2, hw2, b2))

    hw3 = (h2 @ W3)[:, 0]
    p3 = _scalar_conv(edata, hw3 * dinv)
    c = (p3[0] + p3[1]) * dinv + hw3 * dinv2 + b3[0]

    choice = jax.nn.softmax(c, axis=0)
    v = jnp.mean(h2, axis=0, keepdims=True)
    value = (v @ A2w.T + A2b).squeeze()
    return (choice, value)


# speedup vs baseline: 33.5810x; 1.0222x over previous
import jax
import jax.numpy as jnp
from jax import lax
from jax.experimental import pallas as pl
from jax.experimental.pallas import tpu as pltpu
from jax.experimental.pallas import tpu_sc as plsc

N = 10000
D = 128
E = 320000
H = 16

NC = 2
NS = 16
NW = NC * NS
CHUNK = 512
EPT = 10240
NCH = EPT // CHUNK
EPAD = NW * EPT
SLAB = 640
SLAB_LAST = N - 15 * SLAB

_SC_PARAMS = pltpu.CompilerParams(
    needs_layout_passes=False, use_tc_tiling_on_sc=False)


def _row_conv_body(row_hbm, col_hbm, w_hbm, table_hbm, out_hbm,
                   acc_sh, eb0, eb1, msg0, msg1, slab_v,
                   sem_e0, sem_e1, sem_g0, sem_g1):
    c = lax.axis_index("c")
    s = lax.axis_index("s")
    ebase = (c * NS + s) * EPT

    def zbody(i, _):
        slab_v[i, :] = jnp.zeros((H,), jnp.float32)
        return 0
    lax.fori_loop(0, SLAB, zbody, 0)

    @pl.when(s < 15)
    def _():
        pltpu.sync_copy(slab_v, acc_sh.at[pl.ds(s * SLAB, SLAB)])

    @pl.when(s == 15)
    def _():
        pltpu.sync_copy(slab_v.at[pl.ds(0, SLAB_LAST)],
                        acc_sh.at[pl.ds(15 * SLAB, SLAB_LAST)])
    plsc.subcore_barrier()

    iota16 = lax.iota(jnp.int32, 16)

    def start_e(ci, eb, sem):
        base = pl.multiple_of(ebase + ci * CHUNK, 8)
        pltpu.async_copy(row_hbm.at[pl.ds(base, CHUNK)], eb[0], sem)
        pltpu.async_copy(col_hbm.at[pl.ds(base, CHUNK)], eb[1], sem)
        pltpu.async_copy(w_hbm.at[pl.ds(base, CHUNK)], eb[2], sem)

    def wait_e(ci, eb, sem):
        base = pl.multiple_of(ebase + ci * CHUNK, 8)
        pltpu.make_async_copy(row_hbm.at[pl.ds(base, CHUNK)], eb[0], sem).wait()
        pltpu.make_async_copy(col_hbm.at[pl.ds(base, CHUNK)], eb[1], sem).wait()
        pltpu.make_async_copy(w_hbm.at[pl.ds(base, CHUNK)], eb[2], sem).wait()

    def start_g(eb, msg, sem):
        pltpu.async_copy(table_hbm.at[eb[0]], msg, sem)

    def wait_g(eb, msg, sem):
        pltpu.make_async_copy(table_hbm.at[eb[0]], msg, sem).wait()

    def scale_scatter(eb, msg):
        w_v = eb[2]

        def sbody(g, _):
            wv = w_v[pl.ds(g * 16, 16)]
            ev = iota16 + g * 16
            for h in range(H):
                hv = jnp.full((16,), h, jnp.int32)
                v = plsc.load_gather(msg, [ev, hv])
                plsc.store_scatter(msg, [ev, hv], v * wv)
            return 0
        lax.fori_loop(0, CHUNK // 16, sbody, 0)
        pltpu.sync_copy(msg, acc_sh.at[eb[1]], add=True)

    start_e(0, eb0, sem_e0)
    wait_e(0, eb0, sem_e0)
    start_g(eb0, msg0, sem_g0)
    start_e(1, eb1, sem_e1)

    def body(j, _):
        i = 2 * j
        wait_e(i + 1, eb1, sem_e1)
        start_g(eb1, msg1, sem_g1)
        wait_g(eb0, msg0, sem_g0)
        scale_scatter(eb0, msg0)

        @pl.when(i + 2 < NCH)
        def _():
            start_e(i + 2, eb0, sem_e0)

        @pl.when(i + 2 < NCH)
        def _():
            wait_e(i + 2, eb0, sem_e0)
            start_g(eb0, msg0, sem_g0)
        wait_g(eb1, msg1, sem_g1)
        scale_scatter(eb1, msg1)

        @pl.when(i + 3 < NCH)
        def _():
            start_e(i + 3, eb1, sem_e1)
        return 0

    lax.fori_loop(0, NCH // 2, body, 0)
    plsc.subcore_barrier()

    @pl.when(s < 15)
    def _():
        pltpu.sync_copy(acc_sh.at[pl.ds(s * SLAB, SLAB)], slab_v)
        pltpu.sync_copy(slab_v, out_hbm.at[c, pl.ds(s * SLAB, SLAB)])

    @pl.when(s == 15)
    def _():
        pltpu.sync_copy(acc_sh.at[pl.ds(15 * SLAB, SLAB_LAST)],
                        slab_v.at[pl.ds(0, SLAB_LAST)])
        pltpu.sync_copy(slab_v.at[pl.ds(0, SLAB_LAST)],
                        out_hbm.at[c, pl.ds(15 * SLAB, SLAB_LAST)])


def _row_conv(rowp, colp, wp, table):
    mesh = plsc.VectorSubcoreMesh(core_axis_name="c", subcore_axis_name="s")
    f = pl.kernel(
        _row_conv_body,
        out_type=jax.ShapeDtypeStruct((NC, N, H), jnp.float32),
        mesh=mesh,
        compiler_params=_SC_PARAMS,
        scratch_types=[
            pltpu.VMEM_SHARED((N, H), jnp.float32),
            [pltpu.VMEM((CHUNK,), jnp.int32),
             pltpu.VMEM((CHUNK,), jnp.int32),
             pltpu.VMEM((CHUNK,), jnp.float32)],
            [pltpu.VMEM((CHUNK,), jnp.int32),
             pltpu.VMEM((CHUNK,), jnp.int32),
             pltpu.VMEM((CHUNK,), jnp.float32)],
            pltpu.VMEM((CHUNK, H), jnp.float32),
            pltpu.VMEM((CHUNK, H), jnp.float32),
            pltpu.VMEM((SLAB, H), jnp.float32),
            pltpu.SemaphoreType.DMA,
            pltpu.SemaphoreType.DMA,
            pltpu.SemaphoreType.DMA,
            pltpu.SemaphoreType.DMA,
        ],
    )
    return f(rowp, colp, wp, table)


def _scalar_conv_body(row_hbm, col_hbm, w_hbm, table_hbm, out_hbm,
                      acc_sh, tbl_v, eb0, eb1, res0, res1, full_v,
                      sem_e0, sem_e1):
    c = lax.axis_index("c")
    s = lax.axis_index("s")
    ebase = (c * NS + s) * EPT

    pltpu.sync_copy(table_hbm, tbl_v)

    @pl.when(s == 0)
    def _():
        def zbody(i, _):
            full_v[pl.ds(i * 16, 16)] = jnp.zeros((16,), jnp.float32)
            return 0
        lax.fori_loop(0, N // 16, zbody, 0)
        pltpu.sync_copy(full_v, acc_sh)
    plsc.subcore_barrier()

    def start_e(ci, eb, sem):
        base = pl.multiple_of(ebase + ci * CHUNK, 8)
        pltpu.async_copy(row_hbm.at[pl.ds(base, CHUNK)], eb[0], sem)
        pltpu.async_copy(col_hbm.at[pl.ds(base, CHUNK)], eb[1], sem)
        pltpu.async_copy(w_hbm.at[pl.ds(base, CHUNK)], eb[2], sem)

    def wait_e(ci, eb, sem):
        base = pl.multiple_of(ebase + ci * CHUNK, 8)
        pltpu.make_async_copy(row_hbm.at[pl.ds(base, CHUNK)], eb[0], sem).wait()
        pltpu.make_async_copy(col_hbm.at[pl.ds(base, CHUNK)], eb[1], sem).wait()
        pltpu.make_async_copy(w_hbm.at[pl.ds(base, CHUNK)], eb[2], sem).wait()

    def compute_scatter(eb, res):
        def sbody(g, _):
            rv = eb[0][pl.ds(g * 16, 16)]
            wv = eb[2][pl.ds(g * 16, 16)]
            res[pl.ds(g * 16, 16)] = plsc.load_gather(tbl_v, [rv]) * wv
            return 0
        lax.fori_loop(0, CHUNK // 16, sbody, 0)
        pltpu.sync_copy(res, acc_sh.at[eb[1]], add=True)

    start_e(0, eb0, sem_e0)

    def body(j, _):
        i = 2 * j
        wait_e(i, eb0, sem_e0)
        start_e(i + 1, eb1, sem_e1)
        compute_scatter(eb0, res0)
        wait_e(i + 1, eb1, sem_e1)

        @pl.when(i + 2 < NCH)
        def _():
            start_e(i + 2, eb0, sem_e0)
        compute_scatter(eb1, res1)
        return 0

    lax.fori_loop(0, NCH // 2, body, 0)
    plsc.subcore_barrier()

    @pl.when(s == 0)
    def _():
        pltpu.sync_copy(acc_sh, full_v)
        pltpu.sync_copy(full_v, out_hbm.at[c])


def _scalar_conv(rowp, colp, wp, table):
    mesh = plsc.VectorSubcoreMesh(core_axis_name="c", subcore_axis_name="s")
    f = pl.kernel(
        _scalar_conv_body,
        out_type=jax.ShapeDtypeStruct((NC, N), jnp.float32),
        mesh=mesh,
        compiler_params=_SC_PARAMS,
        scratch_types=[
            pltpu.VMEM_SHARED((N,), jnp.float32),
            pltpu.VMEM((N,), jnp.float32),
            [pltpu.VMEM((CHUNK,), jnp.int32),
             pltpu.VMEM((CHUNK,), jnp.int32),
             pltpu.VMEM((CHUNK,), jnp.float32)],
            [pltpu.VMEM((CHUNK,), jnp.int32),
             pltpu.VMEM((CHUNK,), jnp.int32),
             pltpu.VMEM((CHUNK,), jnp.float32)],
            pltpu.VMEM((CHUNK,), jnp.float32),
            pltpu.VMEM((CHUNK,), jnp.float32),
            pltpu.VMEM((N,), jnp.float32),
            pltpu.SemaphoreType.DMA,
            pltpu.SemaphoreType.DMA,
        ],
    )
    return f(rowp, colp, wp, table)


def kernel(x, edge_index, weight, W1, b1, W2, b2, W3, b3, A2w, A2b):
    pad = EPAD - E
    rowp = jnp.concatenate([edge_index[0], jnp.zeros((pad,), jnp.int32)])
    colp = jnp.concatenate([edge_index[1], jnp.zeros((pad,), jnp.int32)])
    wp = jnp.concatenate([weight, jnp.zeros((pad,), jnp.float32)])

    ones_n = jnp.ones((N,), jnp.float32)

    degp = _scalar_conv(rowp, colp, wp, ones_n)
    deg = degp[0] + degp[1] + 1.0
    dinv = jnp.where(deg > 0, deg ** -0.5, 0.0)
    dinv2 = dinv * dinv

    def conv_epilogue(partials, hw, b):
        return (partials[0] + partials[1]) * dinv[:, None] \
            + hw * dinv2[:, None] + b

    hw1 = x @ W1
    p1 = _row_conv(rowp, colp, wp, hw1 * dinv[:, None])
    h1 = jax.nn.relu(conv_epilogue(p1, hw1, b1))

    hw2 = h1 @ W2
    p2 = _row_conv(rowp, colp, wp, hw2 * dinv[:, None])
    h2 = jax.nn.relu(conv_epilogue(p2, hw2, b2))

    hw3 = (h2 @ W3)[:, 0]
    p3 = _scalar_conv(rowp, colp, wp, hw3 * dinv)
    c = (p3[0] + p3[1]) * dinv + hw3 * dinv2 + b3[0]

    choice = jax.nn.softmax(c, axis=0)
    v = jnp.mean(h2, axis=0, keepdims=True)
    value = (v @ A2w.T + A2b).squeeze()
    return (choice, value)


# speedup vs baseline: 33.8333x; 1.0075x over previous
import jax
import jax.numpy as jnp
from jax import lax
from jax.experimental import pallas as pl
from jax.experimental.pallas import tpu as pltpu
from jax.experimental.pallas import tpu_sc as plsc

N = 10000
D = 128
E = 320000
H = 16

NC = 2
NS = 16
NW = NC * NS
CHUNK = 512
EPT = 10240
NCH = EPT // CHUNK
EPAD = NW * EPT
SLAB = 640
SLAB_LAST = N - 15 * SLAB

_SC_PARAMS = pltpu.CompilerParams(
    needs_layout_passes=False, use_tc_tiling_on_sc=False)


def _row_conv_body(row_hbm, col_hbm, w_hbm, table_hbm, out_hbm,
                   acc_sh, eb0, eb1, msg0, msg1, slab_v,
                   sem_e0, sem_e1, sem_g0, sem_g1):
    c = lax.axis_index("c")
    s = lax.axis_index("s")
    ebase = (c * NS + s) * EPT

    def zbody(i, _):
        slab_v[i, :] = jnp.zeros((H,), jnp.float32)
        return 0
    lax.fori_loop(0, SLAB, zbody, 0)

    @pl.when(s < 15)
    def _():
        pltpu.sync_copy(slab_v, acc_sh.at[pl.ds(s * SLAB, SLAB)])

    @pl.when(s == 15)
    def _():
        pltpu.sync_copy(slab_v.at[pl.ds(0, SLAB_LAST)],
                        acc_sh.at[pl.ds(15 * SLAB, SLAB_LAST)])
    plsc.subcore_barrier()

    iota16 = lax.iota(jnp.int32, 16)

    def start_e(ci, eb, sem):
        base = pl.multiple_of(ebase + ci * CHUNK, 8)
        pltpu.async_copy(row_hbm.at[pl.ds(base, CHUNK)], eb[0], sem)
        pltpu.async_copy(col_hbm.at[pl.ds(base, CHUNK)], eb[1], sem)
        pltpu.async_copy(w_hbm.at[pl.ds(base, CHUNK)], eb[2], sem)

    def wait_e(ci, eb, sem):
        base = pl.multiple_of(ebase + ci * CHUNK, 8)
        pltpu.make_async_copy(row_hbm.at[pl.ds(base, CHUNK)], eb[0], sem).wait()
        pltpu.make_async_copy(col_hbm.at[pl.ds(base, CHUNK)], eb[1], sem).wait()
        pltpu.make_async_copy(w_hbm.at[pl.ds(base, CHUNK)], eb[2], sem).wait()

    def start_g(eb, msg, sem):
        pltpu.async_copy(table_hbm.at[eb[0]], msg, sem)

    def wait_g(eb, msg, sem):
        pltpu.make_async_copy(table_hbm.at[eb[0]], msg, sem).wait()

    def scale_scatter(eb, msg):
        w_v = eb[2]

        def sbody(g, _):
            wv = w_v[pl.ds(g * 16, 16)]
            ev = iota16 + g * 16
            for h in range(H):
                hv = jnp.full((16,), h, jnp.int32)
                v = plsc.load_gather(msg, [ev, hv])
                plsc.store_scatter(msg, [ev, hv], v * wv)
            return 0
        lax.fori_loop(0, CHUNK // 16, sbody, 0)
        pltpu.sync_copy(msg, acc_sh.at[eb[1]], add=True)

    start_e(0, eb0, sem_e0)
    wait_e(0, eb0, sem_e0)
    start_g(eb0, msg0, sem_g0)
    start_e(1, eb1, sem_e1)

    def body(j, _):
        i = 2 * j
        wait_e(i + 1, eb1, sem_e1)
        start_g(eb1, msg1, sem_g1)
        wait_g(eb0, msg0, sem_g0)
        scale_scatter(eb0, msg0)

        @pl.when(i + 2 < NCH)
        def _():
            start_e(i + 2, eb0, sem_e0)

        @pl.when(i + 2 < NCH)
        def _():
            wait_e(i + 2, eb0, sem_e0)
            start_g(eb0, msg0, sem_g0)
        wait_g(eb1, msg1, sem_g1)
        scale_scatter(eb1, msg1)

        @pl.when(i + 3 < NCH)
        def _():
            start_e(i + 3, eb1, sem_e1)
        return 0

    lax.fori_loop(0, NCH // 2, body, 0)
    plsc.subcore_barrier()

    @pl.when(s < 15)
    def _():
        pltpu.sync_copy(acc_sh.at[pl.ds(s * SLAB, SLAB)], slab_v)
        pltpu.sync_copy(slab_v, out_hbm.at[c, pl.ds(s * SLAB, SLAB)])

    @pl.when(s == 15)
    def _():
        pltpu.sync_copy(acc_sh.at[pl.ds(15 * SLAB, SLAB_LAST)],
                        slab_v.at[pl.ds(0, SLAB_LAST)])
        pltpu.sync_copy(slab_v.at[pl.ds(0, SLAB_LAST)],
                        out_hbm.at[c, pl.ds(15 * SLAB, SLAB_LAST)])


def _row_conv(rowp, colp, wp, table):
    mesh = plsc.VectorSubcoreMesh(core_axis_name="c", subcore_axis_name="s")
    f = pl.kernel(
        _row_conv_body,
        out_type=jax.ShapeDtypeStruct((NC, N, H), jnp.float32),
        mesh=mesh,
        compiler_params=_SC_PARAMS,
        scratch_types=[
            pltpu.VMEM_SHARED((N, H), jnp.float32),
            [pltpu.VMEM((CHUNK,), jnp.int32),
             pltpu.VMEM((CHUNK,), jnp.int32),
             pltpu.VMEM((CHUNK,), jnp.float32)],
            [pltpu.VMEM((CHUNK,), jnp.int32),
             pltpu.VMEM((CHUNK,), jnp.int32),
             pltpu.VMEM((CHUNK,), jnp.float32)],
            pltpu.VMEM((CHUNK, H), jnp.float32),
            pltpu.VMEM((CHUNK, H), jnp.float32),
            pltpu.VMEM((SLAB, H), jnp.float32),
            pltpu.SemaphoreType.DMA,
            pltpu.SemaphoreType.DMA,
            pltpu.SemaphoreType.DMA,
            pltpu.SemaphoreType.DMA,
        ],
    )
    return f(rowp, colp, wp, table)


def _scalar_conv_body(row_hbm, col_hbm, w_hbm, table_hbm, out_hbm,
                      acc_sh, tbl_v, eb0, eb1, res0, res1, full_v,
                      sem_e0, sem_e1):
    c = lax.axis_index("c")
    s = lax.axis_index("s")
    ebase = (c * NS + s) * EPT

    pltpu.sync_copy(table_hbm, tbl_v)

    @pl.when(s == 0)
    def _():
        def zbody(i, _):
            full_v[pl.ds(i * 16, 16)] = jnp.zeros((16,), jnp.float32)
            return 0
        lax.fori_loop(0, N // 16, zbody, 0)
        pltpu.sync_copy(full_v, acc_sh)
    plsc.subcore_barrier()

    def start_e(ci, eb, sem):
        base = pl.multiple_of(ebase + ci * CHUNK, 8)
        pltpu.async_copy(row_hbm.at[pl.ds(base, CHUNK)], eb[0], sem)
        pltpu.async_copy(col_hbm.at[pl.ds(base, CHUNK)], eb[1], sem)
        pltpu.async_copy(w_hbm.at[pl.ds(base, CHUNK)], eb[2], sem)

    def wait_e(ci, eb, sem):
        base = pl.multiple_of(ebase + ci * CHUNK, 8)
        pltpu.make_async_copy(row_hbm.at[pl.ds(base, CHUNK)], eb[0], sem).wait()
        pltpu.make_async_copy(col_hbm.at[pl.ds(base, CHUNK)], eb[1], sem).wait()
        pltpu.make_async_copy(w_hbm.at[pl.ds(base, CHUNK)], eb[2], sem).wait()

    def compute_scatter(eb, res):
        def sbody(g, _):
            rv = eb[0][pl.ds(g * 16, 16)]
            wv = eb[2][pl.ds(g * 16, 16)]
            res[pl.ds(g * 16, 16)] = plsc.load_gather(tbl_v, [rv]) * wv
            return 0
        lax.fori_loop(0, CHUNK // 16, sbody, 0)
        pltpu.sync_copy(res, acc_sh.at[eb[1]], add=True)

    start_e(0, eb0, sem_e0)

    def body(j, _):
        i = 2 * j
        wait_e(i, eb0, sem_e0)
        start_e(i + 1, eb1, sem_e1)
        compute_scatter(eb0, res0)
        wait_e(i + 1, eb1, sem_e1)

        @pl.when(i + 2 < NCH)
        def _():
            start_e(i + 2, eb0, sem_e0)
        compute_scatter(eb1, res1)
        return 0

    lax.fori_loop(0, NCH // 2, body, 0)
    plsc.subcore_barrier()

    @pl.when(s == 0)
    def _():
        pltpu.sync_copy(acc_sh, full_v)
        pltpu.sync_copy(full_v, out_hbm.at[c])


def _scalar_conv(rowp, colp, wp, table):
    mesh = plsc.VectorSubcoreMesh(core_axis_name="c", subcore_axis_name="s")
    f = pl.kernel(
        _scalar_conv_body,
        out_type=jax.ShapeDtypeStruct((NC, N), jnp.float32),
        mesh=mesh,
        compiler_params=_SC_PARAMS,
        scratch_types=[
            pltpu.VMEM_SHARED((N,), jnp.float32),
            pltpu.VMEM((N,), jnp.float32),
            [pltpu.VMEM((CHUNK,), jnp.int32),
             pltpu.VMEM((CHUNK,), jnp.int32),
             pltpu.VMEM((CHUNK,), jnp.float32)],
            [pltpu.VMEM((CHUNK,), jnp.int32),
             pltpu.VMEM((CHUNK,), jnp.int32),
             pltpu.VMEM((CHUNK,), jnp.float32)],
            pltpu.VMEM((CHUNK,), jnp.float32),
            pltpu.VMEM((CHUNK,), jnp.float32),
            pltpu.VMEM((N,), jnp.float32),
            pltpu.SemaphoreType.DMA,
            pltpu.SemaphoreType.DMA,
        ],
    )
    return f(rowp, colp, wp, table)


def kernel(x, edge_index, weight, W1, b1, W2, b2, W3, b3, A2w, A2b):
    pad = EPAD - E
    rowp = jnp.concatenate([edge_index[0], jnp.zeros((pad,), jnp.int32)])
    colp = jnp.concatenate([edge_index[1], jnp.zeros((pad,), jnp.int32)])
    wp = jnp.concatenate([weight, jnp.zeros((pad,), jnp.float32)])

    ones_n = jnp.ones((N,), jnp.float32)

    degp = _scalar_conv(rowp, colp, wp, ones_n)
    deg = degp[0] + degp[1] + 1.0
    dinv = jnp.where(deg > 0, deg ** -0.5, 0.0)
    dinv2 = dinv * dinv

    NP = N // 8
    eye8 = jnp.eye(8, dtype=jnp.float32)
    x8 = x.reshape(NP, 8 * D)
    dv3 = dinv.reshape(NP, 8, 1)
    dv23 = dinv2.reshape(NP, 8, 1)

    def epilogue(p, hwp, b):
        a = (p[0] + p[1]).reshape(NP, 8, H)
        hw3d = hwp.reshape(NP, 8, H)
        return jax.nn.relu(a * dv3 + hw3d * dv23 + b).reshape(NP, 8 * H)

    hw1p = x8 @ jnp.kron(eye8, W1)
    tbl1 = (hw1p.reshape(NP, 8, H) * dv3).reshape(N, H)
    p1 = _row_conv(rowp, colp, wp, tbl1)
    h1p = epilogue(p1.reshape(NC, NP, 8 * H), hw1p, b1)

    hw2p = h1p @ jnp.kron(eye8, W2)
    tbl2 = (hw2p.reshape(NP, 8, H) * dv3).reshape(N, H)
    p2 = _row_conv(rowp, colp, wp, tbl2)
    h2p = epilogue(p2.reshape(NC, NP, 8 * H), hw2p, b2)

    hw3 = (h2p @ jnp.kron(eye8, W3)).reshape(N)
    p3 = _scalar_conv(rowp, colp, wp, hw3 * dinv)
    c = (p3[0] + p3[1]) * dinv + hw3 * dinv2 + b3[0]

    choice = jax.nn.softmax(c, axis=0)
    v16 = jnp.sum(h2p, axis=0).reshape(8, H).sum(axis=0) / N
    value = jnp.dot(v16, A2w[0]) + A2b[0]
    return (choice, value)
